# baseline (speedup 1.0000x reference)
import functools

import jax
import jax.numpy as jnp
from jax import lax
from jax.experimental import pallas as pl
from jax.experimental.pallas import tpu as pltpu
from jax.experimental.pallas import tpu_sc as plsc

B_ = 16
C_ = 6
L_ = 4096
UPS_ = 5
N_ = B_ * L_


def _decode_body(samp_ref, idx_ref):
    best = samp_ref[:, 0, :]
    bidx = jnp.zeros((B_, L_), jnp.int32)
    for c in range(1, C_):
        val = samp_ref[:, c, :]
        m = val > best
        best = jnp.where(m, val, best)
        bidx = jnp.where(m, jnp.int32(c), bidx)

    prev = jnp.concatenate(
        [jnp.zeros((B_, 1), jnp.int32), bidx[:, : L_ - 1]], axis=1)
    upd = (bidx != 0) & (bidx != prev)
    n = jnp.where(upd, 1, 0).astype(jnp.int32)
    v = jnp.where(upd, bidx - 1, 0).astype(jnp.int32)

    d = 1
    while d < L_:
        z = jnp.zeros((B_, d), jnp.int32)
        na = jnp.concatenate([z, n[:, : L_ - d]], axis=1)
        va = jnp.concatenate([z, v[:, : L_ - d]], axis=1)
        v = ((va << (n + n + n)) | v) & 0x1FF
        n = jnp.minimum(na + n, 3)
        d *= 2

    code = 25 * (v >> 6) + 5 * ((v >> 3) & 7) + (v & 7)
    idx_ref[...] = jnp.where(n < 3, 0, code + 1)


def _lane_iota():
    return lax.iota(jnp.int32, 16)


@functools.cache
def _sc_meshinfo():
    info = plsc.get_sparse_core_info()
    nc, ns = info.num_cores, info.num_subcores
    mesh = plsc.VectorSubcoreMesh(core_axis_name="c", subcore_axis_name="s")
    return nc, ns, mesh


@functools.cache
def _make_sc_gather():
    nc, ns, mesh = _sc_meshinfo()
    nw = nc * ns
    chunk = N_ // nw

    seg = L_ // chunk

    @functools.partial(
        pl.kernel,
        mesh=mesh,
        out_type=(
            jax.ShapeDtypeStruct((N_,), jnp.float32),
            jax.ShapeDtypeStruct((nw, 16), jnp.float32),
        ),
        scratch_types=[
            pltpu.VMEM((256,), jnp.float32),
            pltpu.VMEM((chunk,), jnp.int32),
            pltpu.VMEM((chunk,), jnp.float32),
            pltpu.VMEM((16,), jnp.float32),
        ],
        compiler_params=pltpu.CompilerParams(needs_layout_passes=False),
    )
    def gather_k(idx_hbm, tab_hbm, out_hbm, part_hbm, tab_v, idx_v, rc_v, p_v):
        wid = lax.axis_index("s") * nc + lax.axis_index("c")
        base = wid * chunk
        pltpu.sync_copy(tab_hbm, tab_v)
        pltpu.sync_copy(
            idx_hbm.at[wid // seg, pl.ds((wid % seg) * chunk, chunk)], idx_v)

        zero = jnp.zeros((16,), jnp.float32)

        def body(i, carry):
            s, q = carry
            off = i * 16
            ids = idx_v[pl.ds(off, 16)]
            r = plsc.load_gather(tab_v, [ids])
            rc_v[pl.ds(off, 16)] = r
            return s + r, q + r * r

        s, q = lax.fori_loop(0, chunk // 16, body, (zero, zero), unroll=16)
        lane = _lane_iota()
        sv = jnp.broadcast_to(jnp.sum(s), (16,))
        qv = jnp.broadcast_to(jnp.sum(q), (16,))
        p_v[...] = jnp.where(lane == 0, sv, jnp.where(lane == 1, qv, 0.0))
        pltpu.sync_copy(rc_v, out_hbm.at[pl.ds(base, chunk)])
        pltpu.sync_copy(p_v, part_hbm.at[wid])

    return gather_k


@functools.cache
def _make_sc_bn_upsample():
    nc, ns, mesh = _sc_meshinfo()
    nw = nc * ns
    chunk = N_ // nw
    ochunk = chunk * UPS_

    seg = L_ // chunk

    @functools.partial(
        pl.kernel,
        mesh=mesh,
        out_type=jax.ShapeDtypeStruct((N_ * UPS_,), jnp.float32),
        scratch_types=[
            pltpu.VMEM((nw, 16), jnp.float32),
            pltpu.VMEM((16,), jnp.float32),
            pltpu.VMEM((chunk,), jnp.float32),
            pltpu.VMEM((ochunk,), jnp.float32),
        ],
        compiler_params=pltpu.CompilerParams(needs_layout_passes=False),
    )
    def bn_k(rc_hbm, part_hbm, gb_hbm, out_hbm, part_v, gb_v, rc_v, out_v):
        wid = lax.axis_index("s") * nc + lax.axis_index("c")
        base = wid * chunk
        pltpu.sync_copy(part_hbm, part_v)
        pltpu.sync_copy(gb_hbm, gb_v)
        pltpu.sync_copy(rc_hbm.at[pl.ds(base, chunk)], rc_v)

        acc = jnp.zeros((16,), jnp.float32)
        for i in range(nw):
            acc = acc + part_v[i, :]
        inv_n = 1.0 / N_
        mean = jnp.sum(jnp.where(_lane_iota() == 0, acc, 0.0)) * inv_n
        sumsq = jnp.sum(jnp.where(_lane_iota() == 1, acc, 0.0))
        var = sumsq * inv_n - mean * mean
        x = jnp.broadcast_to(var + 1e-5, (16,))
        y0 = plsc.bitcast(0x5F3759DF - (plsc.bitcast(x, jnp.int32) >> 1),
                          jnp.float32)
        for _ in range(3):
            y0 = y0 * (1.5 - 0.5 * x * y0 * y0)
        gb = gb_v[...]
        gamma = jnp.broadcast_to(jnp.sum(jnp.where(_lane_iota() == 0, gb, 0.0)), (16,))
        beta = jnp.broadcast_to(jnp.sum(jnp.where(_lane_iota() == 1, gb, 0.0)), (16,))
        a = gamma * y0
        b = beta - a * mean
        lane = _lane_iota()

        def body(i, carry):
            off = i * 16
            y = a * rc_v[pl.ds(off, 16)] + b
            oid = (lane + off) * UPS_
            for j in range(UPS_):
                plsc.store_scatter(out_v, [oid + j], y)
            return carry

        lax.fori_loop(0, chunk // 16, body, 0, unroll=8)
        pltpu.sync_copy(out_v, out_hbm.at[pl.ds(wid * ochunk, ochunk)])

    return bn_k


@jax.jit
def kernel(sampling, emb_table, bn_gamma, bn_beta):
    idx = pl.pallas_call(
        _decode_body,
        out_shape=jax.ShapeDtypeStruct((B_, L_), jnp.int32),
        compiler_params=pltpu.CompilerParams(allow_input_fusion=[True]),
    )(sampling)

    rc, part = _make_sc_gather()(idx, emb_table[:, 0])

    gb = jnp.concatenate(
        [bn_gamma, bn_beta, jnp.zeros((14,), jnp.float32)])
    out = _make_sc_bn_upsample()(rc, part, gb)
    return out.reshape(B_, UPS_ * L_, 1)


# speedup vs baseline: 1.0164x; 1.0164x over previous
import functools

import jax
import jax.numpy as jnp
from jax import lax
from jax.experimental import pallas as pl
from jax.experimental.pallas import tpu as pltpu
from jax.experimental.pallas import tpu_sc as plsc

B_ = 16
C_ = 6
L_ = 4096
UPS_ = 5
N_ = B_ * L_


def _decode_body(samp_ref, idx_ref):
    best = samp_ref[:, 0, :]
    bidx = jnp.zeros((B_, L_), jnp.int32)
    for c in range(1, C_):
        val = samp_ref[:, c, :]
        m = val > best
        best = jnp.where(m, val, best)
        bidx = jnp.where(m, jnp.int32(c), bidx)

    prev = jnp.concatenate(
        [jnp.zeros((B_, 1), jnp.int32), bidx[:, : L_ - 1]], axis=1)
    upd = (bidx != 0) & (bidx != prev)
    n = jnp.where(upd, 1, 0).astype(jnp.int32)
    v = jnp.where(upd, bidx - 1, 0).astype(jnp.int32)

    d = 1
    while d < L_:
        z = jnp.zeros((B_, d), jnp.int32)
        na = jnp.concatenate([z, n[:, : L_ - d]], axis=1)
        va = jnp.concatenate([z, v[:, : L_ - d]], axis=1)
        v = ((va << (n + n + n)) | v) & 0x1FF
        n = jnp.minimum(na + n, 3)
        d *= 2

    code = 25 * (v >> 6) + 5 * ((v >> 3) & 7) + (v & 7)
    idx_ref[...] = jnp.where(n < 3, 0, code + 1)


def _lane_iota():
    return lax.iota(jnp.int32, 16)


@functools.cache
def _sc_meshinfo():
    info = plsc.get_sparse_core_info()
    nc, ns = info.num_cores, info.num_subcores
    mesh = plsc.VectorSubcoreMesh(core_axis_name="c", subcore_axis_name="s")
    return nc, ns, mesh


@functools.cache
def _make_sc_gather():
    nc, ns, mesh = _sc_meshinfo()
    nw = nc * ns
    chunk = N_ // nw

    seg = L_ // chunk

    @functools.partial(
        pl.kernel,
        mesh=mesh,
        out_type=(
            jax.ShapeDtypeStruct((N_,), jnp.float32),
            jax.ShapeDtypeStruct((nw, 16), jnp.float32),
        ),
        scratch_types=[
            pltpu.VMEM((256,), jnp.float32),
            pltpu.VMEM((chunk,), jnp.int32),
            pltpu.VMEM((chunk,), jnp.float32),
            pltpu.VMEM((16,), jnp.float32),
        ],
        compiler_params=pltpu.CompilerParams(needs_layout_passes=False),
    )
    def gather_k(idx_hbm, tab_hbm, out_hbm, part_hbm, tab_v, idx_v, rc_v, p_v):
        wid = lax.axis_index("s") * nc + lax.axis_index("c")
        base = wid * chunk
        pltpu.sync_copy(tab_hbm, tab_v)
        pltpu.sync_copy(
            idx_hbm.at[wid // seg, pl.ds((wid % seg) * chunk, chunk)], idx_v)

        zero = jnp.zeros((16,), jnp.float32)

        def body(i, carry):
            s, q = carry
            off = i * 16
            ids = idx_v[pl.ds(off, 16)]
            r = plsc.load_gather(tab_v, [ids])
            rc_v[pl.ds(off, 16)] = r
            return s + r, q + r * r

        s, q = lax.fori_loop(0, chunk // 16, body, (zero, zero), unroll=8)
        lane = _lane_iota()
        sv = jnp.broadcast_to(jnp.sum(s), (16,))
        qv = jnp.broadcast_to(jnp.sum(q), (16,))
        p_v[...] = jnp.where(lane == 0, sv, jnp.where(lane == 1, qv, 0.0))
        pltpu.sync_copy(rc_v, out_hbm.at[pl.ds(base, chunk)])
        pltpu.sync_copy(p_v, part_hbm.at[wid])

    return gather_k


@functools.cache
def _make_sc_bn_upsample():
    nc, ns, mesh = _sc_meshinfo()
    nw = nc * ns
    chunk = N_ // nw
    ochunk = chunk * UPS_

    seg = L_ // chunk

    @functools.partial(
        pl.kernel,
        mesh=mesh,
        out_type=jax.ShapeDtypeStruct((N_ * UPS_,), jnp.float32),
        scratch_types=[
            pltpu.VMEM((nw, 16), jnp.float32),
            pltpu.VMEM((16,), jnp.float32),
            pltpu.VMEM((chunk,), jnp.float32),
            pltpu.VMEM((ochunk,), jnp.float32),
        ],
        compiler_params=pltpu.CompilerParams(needs_layout_passes=False),
    )
    def bn_k(rc_hbm, part_hbm, gb_hbm, out_hbm, part_v, gb_v, rc_v, out_v):
        wid = lax.axis_index("s") * nc + lax.axis_index("c")
        base = wid * chunk
        pltpu.sync_copy(part_hbm, part_v)
        pltpu.sync_copy(gb_hbm, gb_v)
        pltpu.sync_copy(rc_hbm.at[pl.ds(base, chunk)], rc_v)

        acc = jnp.zeros((16,), jnp.float32)
        for i in range(nw):
            acc = acc + part_v[i, :]
        inv_n = 1.0 / N_
        mean = jnp.sum(jnp.where(_lane_iota() == 0, acc, 0.0)) * inv_n
        sumsq = jnp.sum(jnp.where(_lane_iota() == 1, acc, 0.0))
        var = sumsq * inv_n - mean * mean
        x = jnp.broadcast_to(var + 1e-5, (16,))
        y0 = plsc.bitcast(0x5F3759DF - (plsc.bitcast(x, jnp.int32) >> 1),
                          jnp.float32)
        for _ in range(3):
            y0 = y0 * (1.5 - 0.5 * x * y0 * y0)
        gb = gb_v[...]
        gamma = jnp.broadcast_to(jnp.sum(jnp.where(_lane_iota() == 0, gb, 0.0)), (16,))
        beta = jnp.broadcast_to(jnp.sum(jnp.where(_lane_iota() == 1, gb, 0.0)), (16,))
        a = gamma * y0
        b = beta - a * mean
        lane = _lane_iota()

        def body(i, carry):
            off = i * 16
            y = a * rc_v[pl.ds(off, 16)] + b
            oid = (lane + off) * UPS_
            for j in range(UPS_):
                plsc.store_scatter(out_v, [oid + j], y)
            return carry

        lax.fori_loop(0, chunk // 16, body, 0, unroll=4)
        pltpu.sync_copy(out_v, out_hbm.at[pl.ds(wid * ochunk, ochunk)])

    return bn_k


@jax.jit
def kernel(sampling, emb_table, bn_gamma, bn_beta):
    idx = pl.pallas_call(
        _decode_body,
        out_shape=jax.ShapeDtypeStruct((B_, L_), jnp.int32),
        compiler_params=pltpu.CompilerParams(allow_input_fusion=[True]),
    )(sampling)

    rc, part = _make_sc_gather()(idx, emb_table[:, 0])

    gb = jnp.concatenate(
        [bn_gamma, bn_beta, jnp.zeros((14,), jnp.float32)])
    out = _make_sc_bn_upsample()(rc, part, gb)
    return out.reshape(B_, UPS_ * L_, 1)
